# baseline (speedup 1.0000x reference)
import jax
import jax.numpy as jnp
from jax import lax
from jax.experimental import pallas as pl
from jax.experimental.pallas import tpu as pltpu

MARGIN_SQ = 0.25

ROWS, COLS = 16384, 128
TC_BLOCK = 8192
TC_GRID = ROWS // TC_BLOCK


def _tc_body(p_ref, r_ref, out_ref, acc_sq, acc_cnt):
    i = pl.program_id(0)
    p = p_ref[...]
    r = r_ref[...]
    d = r - p
    sq = d * d
    m = sq > MARGIN_SQ
    csq = jnp.sum(jnp.where(m, sq, 0.0).reshape(TC_BLOCK // 8, 8, COLS), axis=0)
    ccnt = jnp.sum(jnp.where(m, 1.0, 0.0).reshape(TC_BLOCK // 8, 8, COLS), axis=0)

    @pl.when(i == 0)
    def _():
        acc_sq[...] = csq
        acc_cnt[...] = ccnt

    @pl.when(i > 0)
    def _():
        acc_sq[...] += csq
        acc_cnt[...] += ccnt

    @pl.when(i == TC_GRID - 1)
    def _():
        s = jnp.sum(acc_sq[...])
        n = jnp.sum(acc_cnt[...])
        out_ref[0, 0] = jnp.where(n > 0.0, s / jnp.maximum(n, 1.0), 0.0)


def kernel(pred, real):
    out = pl.pallas_call(
        _tc_body,
        grid=(TC_GRID,),
        in_specs=[
            pl.BlockSpec((TC_BLOCK, COLS), lambda i: (i, 0)),
            pl.BlockSpec((TC_BLOCK, COLS), lambda i: (i, 0)),
        ],
        out_specs=pl.BlockSpec(memory_space=pltpu.SMEM),
        out_shape=jax.ShapeDtypeStruct((1, 1), jnp.float32),
        scratch_shapes=[
            pltpu.VMEM((8, COLS), jnp.float32),
            pltpu.VMEM((8, COLS), jnp.float32),
        ],
    )(pred, real)
    return out[0, 0]
